# baseline (speedup 1.0000x reference)
import functools

import jax
import jax.numpy as jnp
from jax import lax
from jax.experimental import pallas as pl
from jax.experimental.pallas import tpu as pltpu
from jax.experimental.pallas import tpu_sc as plsc

B = 2
S = 2048
N = B * S
D = 1024
O = 1024
E = 8
H = 512
K = 2

NC = 2
NS = 16
L = 16
NW = NC * NS
TPW = N // NW
NG = N // L
G0G = TPW // L

BLK = 2048
NB = (K * N) // BLK + E
TOTP = NB * BLK
NBA = 48

XCH = 32
CCH = 16
DW = D // 2
OW = O // 2


def _pack_pair(lo_bf, hi_bf):
    lo = lax.convert_element_type(
        lax.bitcast_convert_type(lo_bf, jnp.uint16), jnp.uint32)
    hi = lax.convert_element_type(
        lax.bitcast_convert_type(hi_bf, jnp.uint16), jnp.uint32)
    return lax.bitcast_convert_type(lo | (hi << 16), jnp.int32)


def _unpack_pair(w32):
    u = lax.bitcast_convert_type(w32, jnp.uint32)
    lo = lax.bitcast_convert_type(
        lax.convert_element_type(u & 0xFFFF, jnp.uint16), jnp.bfloat16)
    hi = lax.bitcast_convert_type(
        lax.convert_element_type(u >> 16, jnp.uint16), jnp.bfloat16)
    return lo, hi


def _gate_body(x_ref, wg_ref, o_ref, xp_ref):
    xv = x_ref[...]
    o_ref[...] = lax.dot_general(wg_ref[...], xv,
                                 (((0,), (1,)), ((), ())),
                                 preferred_element_type=jnp.float32)
    xb = xv.astype(jnp.bfloat16)
    xp_ref[...] = _pack_pair(xb[:, :DW], xb[:, DW:])


def _gate(xf, wg):
    return pl.pallas_call(
        _gate_body,
        grid=(N // 512,),
        in_specs=[
            pl.BlockSpec((512, D), lambda i: (i, 0)),
            pl.BlockSpec((D, E), lambda i: (0, 0)),
        ],
        out_specs=[
            pl.BlockSpec((E, 512), lambda i: (0, i)),
            pl.BlockSpec((512, DW), lambda i: (i, 0)),
        ],
        out_shape=[
            jax.ShapeDtypeStruct((E, N), jnp.float32),
            jax.ShapeDtypeStruct((N, DW), jnp.int32),
        ],
    )(xf, wg)


def _wcast_body(w1_ref, w2_ref, o1_ref, o2_ref):
    o1_ref[...] = w1_ref[...].astype(jnp.bfloat16)
    o2_ref[...] = w2_ref[...].astype(jnp.bfloat16)


def _wcast(fc1_w, fc2_w):
    return pl.pallas_call(
        _wcast_body,
        grid=(E,),
        in_specs=[
            pl.BlockSpec((1, H, D), lambda e: (e, 0, 0)),
            pl.BlockSpec((1, O, H), lambda e: (e, 0, 0)),
        ],
        out_specs=[
            pl.BlockSpec((1, H, D), lambda e: (e, 0, 0)),
            pl.BlockSpec((1, O, H), lambda e: (e, 0, 0)),
        ],
        out_shape=[
            jax.ShapeDtypeStruct((E, H, D), jnp.bfloat16),
            jax.ShapeDtypeStruct((E, O, H), jnp.bfloat16),
        ],
    )(fc1_w, fc2_w)


_SC_MESH = plsc.VectorSubcoreMesh(core_axis_name="c", subcore_axis_name="s")


@functools.partial(
    pl.kernel,
    out_type=[
        jax.ShapeDtypeStruct((TOTP, DW), jnp.int32),
        jax.ShapeDtypeStruct((NW, K * 4, XCH), jnp.int32),
        jax.ShapeDtypeStruct((NBA,), jnp.int32),
    ],
    mesh=_SC_MESH,
    compiler_params=pltpu.CompilerParams(needs_layout_passes=False),
    scratch_types=[
        pltpu.VMEM((E, N), jnp.float32),
        pltpu.VMEM((K * TPW,), jnp.int32),
        pltpu.VMEM((K * 4, XCH), jnp.int32),
        pltpu.VMEM((NBA,), jnp.int32),
        pltpu.VMEM((2, XCH, DW), jnp.int32),
        pltpu.SemaphoreType.DMA,
        pltpu.SemaphoreType.DMA,
        pltpu.SemaphoreType.DMA,
    ],
)
def _route_dispatch(gate_hbm, xp_hbm, xs_hbm, dest_hbm, eid_hbm,
                    gate_v, ech_v, destv, eid_v, xbuf, semg, semx, semo):
    cid = lax.axis_index("c")
    sid = lax.axis_index("s")
    wid = sid * NC + cid
    n0 = wid * TPW
    g0 = wid * G0G

    cpg = pltpu.async_copy(gate_hbm, gate_v, semg)

    def load(ch, p):
        return pltpu.async_copy(
            xp_hbm.at[pl.ds(n0 + ch * XCH, XCH), :], xbuf.at[p], semx)

    lds = {0: load(0, 0), 1: load(1, 1)}
    cpg.wait()

    lane = lax.iota(jnp.int32, L)
    lane_is = [lane == e for e in range(E)]
    erow = [jnp.full((L,), e, jnp.int32) for e in range(E)]
    neg = jnp.float32(-3.0e38)

    def group_body(g, carry):
        cnt, pre = carry
        rowb = lane + g * L
        gv = [plsc.load_gather(gate_v, [erow[e], rowb]) for e in range(E)]
        m1 = gv[0]
        i1 = jnp.zeros((L,), jnp.int32)
        for e in range(1, E):
            gt = gv[e] > m1
            m1 = jnp.where(gt, gv[e], m1)
            i1 = jnp.where(gt, e, i1)
        m2 = jnp.where(i1 == 0, neg, gv[0])
        i2 = jnp.zeros((L,), jnp.int32)
        for e in range(1, E):
            ge = jnp.where(i1 == e, neg, gv[e])
            gt = ge > m2
            m2 = jnp.where(gt, ge, m2)
            i2 = jnp.where(gt, e, i2)
        before = g < g0
        for e in range(E):
            ce = (plsc.all_reduce_population_count(i1 == e)
                  + plsc.all_reduce_population_count(i2 == e))
            add = jnp.where(lane_is[e], ce, 0)
            cnt = cnt + add
            pre = pre + jnp.where(before, add, 0)
        own = jnp.logical_and(g >= g0, g < g0 + G0G)

        @pl.when(own)
        def _():
            off = (g - g0) * L
            ech_v[pl.ds(off, L)] = i1
            ech_v[pl.ds(TPW + off, L)] = i2

        return cnt, pre

    zero = jnp.zeros((L,), jnp.int32)
    cnt, pre = lax.fori_loop(0, NG, group_body, (zero, zero))

    lg = BLK.bit_length() - 1
    pad = jnp.left_shift(jnp.right_shift(cnt + (BLK - 1), lg), lg)
    padcum = plsc.cumsum(pad)
    start_pad = padcum - pad
    base = start_pad + pre
    bs = jnp.right_shift(start_pad, lg)

    run = base
    for k in range(K):
        for c in range(G0G):
            a = ech_v[pl.ds(k * TPW + c * L, L)]
            dvec = jnp.zeros((L,), jnp.int32)
            for e in range(E):
                m = a == e
                pc = plsc.cumsum(jnp.where(m, 1, 0))
                dvec = dvec + jnp.where(m, run[e] + pc - 1, 0)
                run = run + jnp.where(lane_is[e], pc[L - 1], 0)
            destv[k * 4 + c // 2, pl.ds((c % 2) * L, L)] = dvec
    pltpu.sync_copy(destv, dest_hbm.at[wid])

    @pl.when(wid == 0)
    def _():
        total_nb = bs[E]
        for j in range(NBA // L):
            bidx = lane + j * L
            ev = jnp.full((L,), -1, jnp.int32)
            for e in range(E):
                ev = ev + jnp.where(bidx >= bs[e], 1, 0)
            ev = jnp.where(bidx < total_nb, ev, -1)
            eid_v[pl.ds(j * L, L)] = ev
        pltpu.sync_copy(eid_v, eid_hbm)

    nch = TPW // XCH
    pend = None
    for ch in range(nch):
        lds[ch].wait()
        s1 = pltpu.async_copy(xbuf.at[ch % 2], xs_hbm.at[destv.at[ch]], semo)
        s2 = pltpu.async_copy(xbuf.at[ch % 2], xs_hbm.at[destv.at[4 + ch]],
                              semo)
        if pend is not None:
            pend[0].wait()
            pend[1].wait()
            if ch + 1 < nch:
                lds[ch + 1] = load(ch + 1, (ch + 1) % 2)
        pend = (s1, s2)
    pend[0].wait()
    pend[1].wait()


def _ffn_body(eid_ref, xs_ref, w1_ref, w2_ref, ys_ref):
    b = pl.program_id(0)

    @pl.when(eid_ref[b] >= 0)
    def _():
        lo, hi = _unpack_pair(xs_ref[...])
        xb = jnp.concatenate([lo, hi], axis=1)
        h = lax.dot_general(xb, w1_ref[0],
                            (((1,), (1,)), ((), ())),
                            preferred_element_type=jnp.float32)
        hb = jnp.maximum(h, 0.0).astype(jnp.bfloat16)
        y = lax.dot_general(hb, w2_ref[0],
                            (((1,), (1,)), ((), ())),
                            preferred_element_type=jnp.float32)
        yb = y.astype(jnp.bfloat16)
        ys_ref[...] = _pack_pair(yb[:, :OW], yb[:, OW:])


def _ffn(eid, xs, w1b, w2b):
    grid_spec = pltpu.PrefetchScalarGridSpec(
        num_scalar_prefetch=1,
        grid=(NB,),
        in_specs=[
            pl.BlockSpec((BLK, DW),
                         lambda b, eid: (jnp.where(eid[b] < 0, NB - 1, b), 0)),
            pl.BlockSpec((1, H, D),
                         lambda b, eid: (jnp.maximum(eid[b], 0), 0, 0)),
            pl.BlockSpec((1, O, H),
                         lambda b, eid: (jnp.maximum(eid[b], 0), 0, 0)),
        ],
        out_specs=pl.BlockSpec(
            (BLK, OW),
            lambda b, eid: (jnp.where(eid[b] < 0, NB - 1, b), 0)),
    )
    return pl.pallas_call(
        _ffn_body,
        grid_spec=grid_spec,
        out_shape=jax.ShapeDtypeStruct((TOTP, OW), jnp.int32),
    )(eid, xs, w1b, w2b)


@functools.partial(
    pl.kernel,
    out_type=jax.ShapeDtypeStruct((N, O), jnp.float32),
    mesh=_SC_MESH,
    compiler_params=pltpu.CompilerParams(needs_layout_passes=False),
    scratch_types=[
        pltpu.VMEM((K * 4, XCH), jnp.int32),
        pltpu.VMEM((2, CCH, OW), jnp.int32),
        pltpu.VMEM((2, CCH, OW), jnp.int32),
        pltpu.VMEM((2, CCH, O), jnp.float32),
        pltpu.SemaphoreType.DMA,
        pltpu.SemaphoreType.DMA,
        pltpu.SemaphoreType.DMA,
    ],
)
def _combine(ys_hbm, dest_hbm, out_hbm, dv, y1, y2, ob, sem1, sem2, semo):
    cid = lax.axis_index("c")
    sid = lax.axis_index("s")
    wid = sid * NC + cid
    n0 = wid * TPW
    nch = TPW // CCH

    pltpu.sync_copy(dest_hbm.at[wid], dv)

    def fire(i):
        ch, half = i // 2, i % 2
        p = i % 2
        g1 = pltpu.async_copy(
            ys_hbm.at[dv.at[ch, pl.ds(half * CCH, CCH)]], y1.at[p], sem1)
        g2 = pltpu.async_copy(
            ys_hbm.at[dv.at[4 + ch, pl.ds(half * CCH, CCH)]], y2.at[p], sem2)
        return g1, g2

    pend = fire(0)
    outw = None
    for i in range(nch):
        p = i % 2
        if outw is not None:
            outw.wait()
        nxt = fire(i + 1) if i + 1 < nch else None
        g1, g2 = pend
        g1.wait()
        g2.wait()

        def addrow(r, _):
            himask = jnp.int32(-65536)
            for g in range(OW // L):
                sl = pl.ds(g * L, L)
                w1v = y1[p, r, sl]
                w2v = y2[p, r, sl]
                lo = (plsc.bitcast(jnp.left_shift(w1v, 16), jnp.float32)
                      + plsc.bitcast(jnp.left_shift(w2v, 16), jnp.float32))
                hi = (plsc.bitcast(w1v & himask, jnp.float32)
                      + plsc.bitcast(w2v & himask, jnp.float32))
                ob[p, r, sl] = lo
                ob[p, r, pl.ds(OW + g * L, L)] = hi
            return 0

        lax.fori_loop(0, CCH, addrow, 0)
        outw = pltpu.async_copy(
            ob.at[p], out_hbm.at[pl.ds(n0 + i * CCH, CCH), :], semo)
        pend = nxt
    outw.wait()


def kernel(x, wg, fc1_w, fc2_w):
    xf = x.reshape(N, D)
    gate, xp = _gate(xf, wg)
    w1b, w2b = _wcast(fc1_w, fc2_w)
    xs, dest, eid = _route_dispatch(gate, xp)
    ys = _ffn(eid, xs, w1b, w2b)
    out = _combine(ys, dest)
    return out.reshape(B, S, O)


# speedup vs baseline: 1.0921x; 1.0921x over previous
import functools

import jax
import jax.numpy as jnp
from jax import lax
from jax.experimental import pallas as pl
from jax.experimental.pallas import tpu as pltpu
from jax.experimental.pallas import tpu_sc as plsc

B = 2
S = 2048
N = B * S
D = 1024
O = 1024
E = 8
H = 512
K = 2

NC = 2
NS = 16
L = 16
NW = NC * NS
TPW = N // NW
NG = N // L
G0G = TPW // L

BLK = 1024
NB = (K * N) // BLK + E
TOTP = NB * BLK
NBA = 48

XCH = 32
CCH = 16
DW = D // 2
OW = O // 2


def _pack_pair(lo_bf, hi_bf):
    lo = lax.convert_element_type(
        lax.bitcast_convert_type(lo_bf, jnp.uint16), jnp.uint32)
    hi = lax.convert_element_type(
        lax.bitcast_convert_type(hi_bf, jnp.uint16), jnp.uint32)
    return lax.bitcast_convert_type(lo | (hi << 16), jnp.int32)


def _unpack_pair(w32):
    u = lax.bitcast_convert_type(w32, jnp.uint32)
    lo = lax.bitcast_convert_type(
        lax.convert_element_type(u & 0xFFFF, jnp.uint16), jnp.bfloat16)
    hi = lax.bitcast_convert_type(
        lax.convert_element_type(u >> 16, jnp.uint16), jnp.bfloat16)
    return lo, hi


def _gate_body(x_ref, wg_ref, o_ref, xp_ref):
    xv = x_ref[...]
    o_ref[...] = lax.dot_general(wg_ref[...], xv,
                                 (((0,), (1,)), ((), ())),
                                 preferred_element_type=jnp.float32)
    xb = xv.astype(jnp.bfloat16)
    xp_ref[...] = _pack_pair(xb[:, :DW], xb[:, DW:])


def _gate(xf, wg):
    return pl.pallas_call(
        _gate_body,
        grid=(N // 512,),
        in_specs=[
            pl.BlockSpec((512, D), lambda i: (i, 0)),
            pl.BlockSpec((D, E), lambda i: (0, 0)),
        ],
        out_specs=[
            pl.BlockSpec((E, 512), lambda i: (0, i)),
            pl.BlockSpec((512, DW), lambda i: (i, 0)),
        ],
        out_shape=[
            jax.ShapeDtypeStruct((E, N), jnp.float32),
            jax.ShapeDtypeStruct((N, DW), jnp.int32),
        ],
    )(xf, wg)


def _wcast_body(w1_ref, w2_ref, o1_ref, o2_ref):
    o1_ref[...] = w1_ref[...].astype(jnp.bfloat16)
    o2_ref[...] = w2_ref[...].astype(jnp.bfloat16)


def _wcast(fc1_w, fc2_w):
    return pl.pallas_call(
        _wcast_body,
        grid=(E,),
        in_specs=[
            pl.BlockSpec((1, H, D), lambda e: (e, 0, 0)),
            pl.BlockSpec((1, O, H), lambda e: (e, 0, 0)),
        ],
        out_specs=[
            pl.BlockSpec((1, H, D), lambda e: (e, 0, 0)),
            pl.BlockSpec((1, O, H), lambda e: (e, 0, 0)),
        ],
        out_shape=[
            jax.ShapeDtypeStruct((E, H, D), jnp.bfloat16),
            jax.ShapeDtypeStruct((E, O, H), jnp.bfloat16),
        ],
    )(fc1_w, fc2_w)


_SC_MESH = plsc.VectorSubcoreMesh(core_axis_name="c", subcore_axis_name="s")


@functools.partial(
    pl.kernel,
    out_type=[
        jax.ShapeDtypeStruct((TOTP, DW), jnp.int32),
        jax.ShapeDtypeStruct((NW, K * 4, XCH), jnp.int32),
        jax.ShapeDtypeStruct((NBA,), jnp.int32),
    ],
    mesh=_SC_MESH,
    compiler_params=pltpu.CompilerParams(needs_layout_passes=False),
    scratch_types=[
        pltpu.VMEM((E, N), jnp.float32),
        pltpu.VMEM((K * TPW,), jnp.int32),
        pltpu.VMEM((K * 4, XCH), jnp.int32),
        pltpu.VMEM((NBA,), jnp.int32),
        pltpu.VMEM((2, XCH, DW), jnp.int32),
        pltpu.SemaphoreType.DMA,
        pltpu.SemaphoreType.DMA,
        pltpu.SemaphoreType.DMA,
    ],
)
def _route_dispatch(gate_hbm, xp_hbm, xs_hbm, dest_hbm, eid_hbm,
                    gate_v, ech_v, destv, eid_v, xbuf, semg, semx, semo):
    cid = lax.axis_index("c")
    sid = lax.axis_index("s")
    wid = sid * NC + cid
    n0 = wid * TPW
    g0 = wid * G0G

    cpg = pltpu.async_copy(gate_hbm, gate_v, semg)

    def load(ch, p):
        return pltpu.async_copy(
            xp_hbm.at[pl.ds(n0 + ch * XCH, XCH), :], xbuf.at[p], semx)

    lds = {0: load(0, 0), 1: load(1, 1)}
    cpg.wait()

    lane = lax.iota(jnp.int32, L)
    lane_is = [lane == e for e in range(E)]
    erow = [jnp.full((L,), e, jnp.int32) for e in range(E)]
    neg = jnp.float32(-3.0e38)

    def group_body(g, carry):
        cnt, pre = carry
        rowb = lane + g * L
        gv = [plsc.load_gather(gate_v, [erow[e], rowb]) for e in range(E)]
        m1 = gv[0]
        i1 = jnp.zeros((L,), jnp.int32)
        for e in range(1, E):
            gt = gv[e] > m1
            m1 = jnp.where(gt, gv[e], m1)
            i1 = jnp.where(gt, e, i1)
        m2 = jnp.where(i1 == 0, neg, gv[0])
        i2 = jnp.zeros((L,), jnp.int32)
        for e in range(1, E):
            ge = jnp.where(i1 == e, neg, gv[e])
            gt = ge > m2
            m2 = jnp.where(gt, ge, m2)
            i2 = jnp.where(gt, e, i2)
        before = g < g0
        for e in range(E):
            ce = (plsc.all_reduce_population_count(i1 == e)
                  + plsc.all_reduce_population_count(i2 == e))
            add = jnp.where(lane_is[e], ce, 0)
            cnt = cnt + add
            pre = pre + jnp.where(before, add, 0)
        own = jnp.logical_and(g >= g0, g < g0 + G0G)

        @pl.when(own)
        def _():
            off = (g - g0) * L
            ech_v[pl.ds(off, L)] = i1
            ech_v[pl.ds(TPW + off, L)] = i2

        return cnt, pre

    zero = jnp.zeros((L,), jnp.int32)
    cnt, pre = lax.fori_loop(0, NG, group_body, (zero, zero))

    lg = BLK.bit_length() - 1
    pad = jnp.left_shift(jnp.right_shift(cnt + (BLK - 1), lg), lg)
    padcum = plsc.cumsum(pad)
    start_pad = padcum - pad
    base = start_pad + pre
    bs = jnp.right_shift(start_pad, lg)

    run = base
    for k in range(K):
        for c in range(G0G):
            a = ech_v[pl.ds(k * TPW + c * L, L)]
            dvec = jnp.zeros((L,), jnp.int32)
            for e in range(E):
                m = a == e
                pc = plsc.cumsum(jnp.where(m, 1, 0))
                dvec = dvec + jnp.where(m, run[e] + pc - 1, 0)
                run = run + jnp.where(lane_is[e], pc[L - 1], 0)
            destv[k * 4 + c // 2, pl.ds((c % 2) * L, L)] = dvec
    pltpu.sync_copy(destv, dest_hbm.at[wid])

    @pl.when(wid == 0)
    def _():
        total_nb = bs[E]
        for j in range(NBA // L):
            bidx = lane + j * L
            ev = jnp.full((L,), -1, jnp.int32)
            for e in range(E):
                ev = ev + jnp.where(bidx >= bs[e], 1, 0)
            ev = jnp.where(bidx < total_nb, ev, -1)
            eid_v[pl.ds(j * L, L)] = ev
        pltpu.sync_copy(eid_v, eid_hbm)

    nch = TPW // XCH
    pend = None
    for ch in range(nch):
        lds[ch].wait()
        s1 = pltpu.async_copy(xbuf.at[ch % 2], xs_hbm.at[destv.at[ch]], semo)
        s2 = pltpu.async_copy(xbuf.at[ch % 2], xs_hbm.at[destv.at[4 + ch]],
                              semo)
        if pend is not None:
            pend[0].wait()
            pend[1].wait()
            if ch + 1 < nch:
                lds[ch + 1] = load(ch + 1, (ch + 1) % 2)
        pend = (s1, s2)
    pend[0].wait()
    pend[1].wait()


def _ffn_body(eid_ref, xs_ref, w1_ref, w2_ref, ys_ref):
    b = pl.program_id(0)

    @pl.when(eid_ref[b] >= 0)
    def _():
        lo, hi = _unpack_pair(xs_ref[...])
        xb = jnp.concatenate([lo, hi], axis=1)
        h = lax.dot_general(xb, w1_ref[0],
                            (((1,), (1,)), ((), ())),
                            preferred_element_type=jnp.float32)
        hb = jnp.maximum(h, 0.0).astype(jnp.bfloat16)
        y = lax.dot_general(hb, w2_ref[0],
                            (((1,), (1,)), ((), ())),
                            preferred_element_type=jnp.float32)
        yb = y.astype(jnp.bfloat16)
        ys_ref[...] = _pack_pair(yb[:, :OW], yb[:, OW:])


def _ffn(eid, xs, w1b, w2b):
    grid_spec = pltpu.PrefetchScalarGridSpec(
        num_scalar_prefetch=1,
        grid=(NB,),
        in_specs=[
            pl.BlockSpec((BLK, DW),
                         lambda b, eid: (jnp.where(eid[b] < 0, NB - 1, b), 0)),
            pl.BlockSpec((1, H, D),
                         lambda b, eid: (jnp.maximum(eid[b], 0), 0, 0)),
            pl.BlockSpec((1, O, H),
                         lambda b, eid: (jnp.maximum(eid[b], 0), 0, 0)),
        ],
        out_specs=pl.BlockSpec(
            (BLK, OW),
            lambda b, eid: (jnp.where(eid[b] < 0, NB - 1, b), 0)),
    )
    return pl.pallas_call(
        _ffn_body,
        grid_spec=grid_spec,
        out_shape=jax.ShapeDtypeStruct((TOTP, OW), jnp.int32),
    )(eid, xs, w1b, w2b)


@functools.partial(
    pl.kernel,
    out_type=jax.ShapeDtypeStruct((N, O), jnp.float32),
    mesh=_SC_MESH,
    compiler_params=pltpu.CompilerParams(needs_layout_passes=False),
    scratch_types=[
        pltpu.VMEM((K * 4, XCH), jnp.int32),
        pltpu.VMEM((2, CCH, OW), jnp.int32),
        pltpu.VMEM((2, CCH, OW), jnp.int32),
        pltpu.VMEM((2, CCH, O), jnp.float32),
        pltpu.SemaphoreType.DMA,
        pltpu.SemaphoreType.DMA,
        pltpu.SemaphoreType.DMA,
    ],
)
def _combine(ys_hbm, dest_hbm, out_hbm, dv, y1, y2, ob, sem1, sem2, semo):
    cid = lax.axis_index("c")
    sid = lax.axis_index("s")
    wid = sid * NC + cid
    n0 = wid * TPW
    nch = TPW // CCH

    pltpu.sync_copy(dest_hbm.at[wid], dv)

    def fire(i):
        ch, half = i // 2, i % 2
        p = i % 2
        g1 = pltpu.async_copy(
            ys_hbm.at[dv.at[ch, pl.ds(half * CCH, CCH)]], y1.at[p], sem1)
        g2 = pltpu.async_copy(
            ys_hbm.at[dv.at[4 + ch, pl.ds(half * CCH, CCH)]], y2.at[p], sem2)
        return g1, g2

    pend = fire(0)
    outw = None
    for i in range(nch):
        p = i % 2
        if outw is not None:
            outw.wait()
        nxt = fire(i + 1) if i + 1 < nch else None
        g1, g2 = pend
        g1.wait()
        g2.wait()

        def addrow(r, _):
            himask = jnp.int32(-65536)
            for g in range(OW // L):
                sl = pl.ds(g * L, L)
                w1v = y1[p, r, sl]
                w2v = y2[p, r, sl]
                lo = (plsc.bitcast(jnp.left_shift(w1v, 16), jnp.float32)
                      + plsc.bitcast(jnp.left_shift(w2v, 16), jnp.float32))
                hi = (plsc.bitcast(w1v & himask, jnp.float32)
                      + plsc.bitcast(w2v & himask, jnp.float32))
                ob[p, r, sl] = lo
                ob[p, r, pl.ds(OW + g * L, L)] = hi
            return 0

        lax.fori_loop(0, CCH, addrow, 0)
        outw = pltpu.async_copy(
            ob.at[p], out_hbm.at[pl.ds(n0 + i * CCH, CCH), :], semo)
        pend = nxt
    outw.wait()


def kernel(x, wg, fc1_w, fc2_w):
    xf = x.reshape(N, D)
    gate, xp = _gate(xf, wg)
    w1b, w2b = _wcast(fc1_w, fc2_w)
    xs, dest, eid = _route_dispatch(gate, xp)
    ys = _ffn(eid, xs, w1b, w2b)
    out = _combine(ys, dest)
    return out.reshape(B, S, O)


# speedup vs baseline: 1.1335x; 1.0379x over previous
import functools

import jax
import jax.numpy as jnp
from jax import lax
from jax.experimental import pallas as pl
from jax.experimental.pallas import tpu as pltpu
from jax.experimental.pallas import tpu_sc as plsc

B = 2
S = 2048
N = B * S
D = 1024
O = 1024
E = 8
H = 512
K = 2

NC = 2
NS = 16
L = 16
NW = NC * NS
TPW = N // NW
NG = N // L
G0G = TPW // L

BLK = 1024
NB = (K * N) // BLK + E
TOTP = NB * BLK
NBA = 48

XCH = 32
CCH = 16
DW = D // 2
OW = O // 2


def _pack_pair(lo_bf, hi_bf):
    lo = lax.convert_element_type(
        lax.bitcast_convert_type(lo_bf, jnp.uint16), jnp.uint32)
    hi = lax.convert_element_type(
        lax.bitcast_convert_type(hi_bf, jnp.uint16), jnp.uint32)
    return lax.bitcast_convert_type(lo | (hi << 16), jnp.int32)


def _unpack_pair(w32):
    u = lax.bitcast_convert_type(w32, jnp.uint32)
    lo = lax.bitcast_convert_type(
        lax.convert_element_type(u & 0xFFFF, jnp.uint16), jnp.bfloat16)
    hi = lax.bitcast_convert_type(
        lax.convert_element_type(u >> 16, jnp.uint16), jnp.bfloat16)
    return lo, hi


def _gate_body(x_ref, wg_ref, o_ref, xp_ref):
    xv = x_ref[...]
    o_ref[...] = lax.dot_general(wg_ref[...], xv,
                                 (((0,), (1,)), ((), ())),
                                 preferred_element_type=jnp.float32)
    xb = xv.astype(jnp.bfloat16)
    xp_ref[...] = _pack_pair(xb[:, :DW], xb[:, DW:])


def _gate(xf, wg):
    return pl.pallas_call(
        _gate_body,
        grid=(N // 512,),
        in_specs=[
            pl.BlockSpec((512, D), lambda i: (i, 0)),
            pl.BlockSpec((D, E), lambda i: (0, 0)),
        ],
        out_specs=[
            pl.BlockSpec((E, 512), lambda i: (0, i)),
            pl.BlockSpec((512, DW), lambda i: (i, 0)),
        ],
        out_shape=[
            jax.ShapeDtypeStruct((E, N), jnp.float32),
            jax.ShapeDtypeStruct((N, DW), jnp.int32),
        ],
    )(xf, wg)


def _wcast_body(w1_ref, w2_ref, o1_ref, o2_ref):
    o1_ref[...] = w1_ref[...].astype(jnp.bfloat16)
    o2_ref[...] = w2_ref[...].astype(jnp.bfloat16)


def _wcast(fc1_w, fc2_w):
    return pl.pallas_call(
        _wcast_body,
        grid=(E,),
        in_specs=[
            pl.BlockSpec((1, H, D), lambda e: (e, 0, 0)),
            pl.BlockSpec((1, O, H), lambda e: (e, 0, 0)),
        ],
        out_specs=[
            pl.BlockSpec((1, H, D), lambda e: (e, 0, 0)),
            pl.BlockSpec((1, O, H), lambda e: (e, 0, 0)),
        ],
        out_shape=[
            jax.ShapeDtypeStruct((E, H, D), jnp.bfloat16),
            jax.ShapeDtypeStruct((E, O, H), jnp.bfloat16),
        ],
    )(fc1_w, fc2_w)


_SC_MESH = plsc.VectorSubcoreMesh(core_axis_name="c", subcore_axis_name="s")


@functools.partial(
    pl.kernel,
    out_type=[
        jax.ShapeDtypeStruct((TOTP, DW), jnp.int32),
        jax.ShapeDtypeStruct((NW, K * 4, XCH), jnp.int32),
        jax.ShapeDtypeStruct((NBA,), jnp.int32),
    ],
    mesh=_SC_MESH,
    compiler_params=pltpu.CompilerParams(needs_layout_passes=False),
    scratch_types=[
        pltpu.VMEM((E, N), jnp.float32),
        pltpu.VMEM((K * TPW,), jnp.int32),
        pltpu.VMEM((K * 4, XCH), jnp.int32),
        pltpu.VMEM((NBA,), jnp.int32),
        pltpu.VMEM((2, XCH, DW), jnp.int32),
        pltpu.SemaphoreType.DMA,
        pltpu.SemaphoreType.DMA,
        pltpu.SemaphoreType.DMA,
    ],
)
def _route_dispatch(gate_hbm, xp_hbm, xs_hbm, dest_hbm, eid_hbm,
                    gate_v, ech_v, destv, eid_v, xbuf, semg, semx, semo):
    cid = lax.axis_index("c")
    sid = lax.axis_index("s")
    wid = sid * NC + cid
    n0 = wid * TPW
    g0 = wid * G0G

    cpg = pltpu.async_copy(gate_hbm, gate_v, semg)

    def load(ch, p):
        return pltpu.async_copy(
            xp_hbm.at[pl.ds(n0 + ch * XCH, XCH), :], xbuf.at[p], semx)

    lds = {0: load(0, 0), 1: load(1, 1)}
    cpg.wait()

    lane = lax.iota(jnp.int32, L)
    lane_is = [lane == e for e in range(E)]
    erow = [jnp.full((L,), e, jnp.int32) for e in range(E)]
    neg = jnp.float32(-3.0e38)

    def group_body(g, carry):
        cnt, pre = carry
        rowb = lane + g * L
        gv = [plsc.load_gather(gate_v, [erow[e], rowb]) for e in range(E)]
        m1 = gv[0]
        i1 = jnp.zeros((L,), jnp.int32)
        for e in range(1, E):
            gt = gv[e] > m1
            m1 = jnp.where(gt, gv[e], m1)
            i1 = jnp.where(gt, e, i1)
        m2 = jnp.where(i1 == 0, neg, gv[0])
        i2 = jnp.zeros((L,), jnp.int32)
        for e in range(1, E):
            ge = jnp.where(i1 == e, neg, gv[e])
            gt = ge > m2
            m2 = jnp.where(gt, ge, m2)
            i2 = jnp.where(gt, e, i2)
        before = g < g0
        for e in range(E):
            ce = (plsc.all_reduce_population_count(i1 == e)
                  + plsc.all_reduce_population_count(i2 == e))
            add = jnp.where(lane_is[e], ce, 0)
            cnt = cnt + add
            pre = pre + jnp.where(before, add, 0)
        own = jnp.logical_and(g >= g0, g < g0 + G0G)

        @pl.when(own)
        def _():
            off = (g - g0) * L
            ech_v[pl.ds(off, L)] = i1
            ech_v[pl.ds(TPW + off, L)] = i2

        return cnt, pre

    zero = jnp.zeros((L,), jnp.int32)
    cnt, pre = lax.fori_loop(0, NG, group_body, (zero, zero))

    lg = BLK.bit_length() - 1
    pad = jnp.left_shift(jnp.right_shift(cnt + (BLK - 1), lg), lg)
    padcum = plsc.cumsum(pad)
    start_pad = padcum - pad
    base = start_pad + pre
    bs = jnp.right_shift(start_pad, lg)

    run = base
    for k in range(K):
        for c in range(G0G):
            a = ech_v[pl.ds(k * TPW + c * L, L)]
            dvec = jnp.zeros((L,), jnp.int32)
            for e in range(E):
                m = a == e
                pc = plsc.cumsum(jnp.where(m, 1, 0))
                dvec = dvec + jnp.where(m, run[e] + pc - 1, 0)
                run = run + jnp.where(lane_is[e], pc[L - 1], 0)
            destv[k * 4 + c // 2, pl.ds((c % 2) * L, L)] = dvec
    pltpu.sync_copy(destv, dest_hbm.at[wid])

    @pl.when(wid == 0)
    def _():
        total_nb = bs[E]
        for j in range(NBA // L):
            bidx = lane + j * L
            ev = jnp.full((L,), -1, jnp.int32)
            for e in range(E):
                ev = ev + jnp.where(bidx >= bs[e], 1, 0)
            ev = jnp.where(bidx < total_nb, ev, -1)
            eid_v[pl.ds(j * L, L)] = ev
        pltpu.sync_copy(eid_v, eid_hbm)

    nch = TPW // XCH
    pend = None
    for ch in range(nch):
        lds[ch].wait()
        s1 = pltpu.async_copy(xbuf.at[ch % 2], xs_hbm.at[destv.at[ch]], semo)
        s2 = pltpu.async_copy(xbuf.at[ch % 2], xs_hbm.at[destv.at[4 + ch]],
                              semo)
        if pend is not None:
            pend[0].wait()
            pend[1].wait()
            if ch + 1 < nch:
                lds[ch + 1] = load(ch + 1, (ch + 1) % 2)
        pend = (s1, s2)
    pend[0].wait()
    pend[1].wait()


def _ffn_body(eid_ref, xs_ref, w1_ref, w2_ref, ys_ref):
    b = pl.program_id(0)

    @pl.when(eid_ref[b] >= 0)
    def _():
        lo, hi = _unpack_pair(xs_ref[...])
        xb = jnp.concatenate([lo, hi], axis=1)
        h = lax.dot_general(xb, w1_ref[0],
                            (((1,), (1,)), ((), ())),
                            preferred_element_type=jnp.float32)
        hb = jnp.maximum(h, 0.0).astype(jnp.bfloat16)
        y = lax.dot_general(hb, w2_ref[0],
                            (((1,), (1,)), ((), ())),
                            preferred_element_type=jnp.float32)
        yb = y.astype(jnp.bfloat16)
        ys_ref[...] = _pack_pair(yb[:, :OW], yb[:, OW:])


def _ffn(eid, xs, w1b, w2b):
    grid_spec = pltpu.PrefetchScalarGridSpec(
        num_scalar_prefetch=1,
        grid=(NB,),
        in_specs=[
            pl.BlockSpec((BLK, DW),
                         lambda b, eid: (jnp.where(eid[b] < 0, NB - 1, b), 0)),
            pl.BlockSpec((1, H, D),
                         lambda b, eid: (jnp.maximum(eid[b], 0), 0, 0)),
            pl.BlockSpec((1, O, H),
                         lambda b, eid: (jnp.maximum(eid[b], 0), 0, 0)),
        ],
        out_specs=pl.BlockSpec(
            (BLK, OW),
            lambda b, eid: (jnp.where(eid[b] < 0, NB - 1, b), 0)),
    )
    return pl.pallas_call(
        _ffn_body,
        grid_spec=grid_spec,
        out_shape=jax.ShapeDtypeStruct((TOTP, OW), jnp.int32),
    )(eid, xs, w1b, w2b)


@functools.partial(
    pl.kernel,
    out_type=jax.ShapeDtypeStruct((N, O), jnp.float32),
    mesh=_SC_MESH,
    compiler_params=pltpu.CompilerParams(needs_layout_passes=False),
    scratch_types=[
        pltpu.VMEM((K * 4, XCH), jnp.int32),
        pltpu.VMEM((4, CCH, OW), jnp.int32),
        pltpu.VMEM((4, CCH, OW), jnp.int32),
        pltpu.VMEM((2, CCH, O), jnp.float32),
        pltpu.SemaphoreType.DMA,
        pltpu.SemaphoreType.DMA,
        pltpu.SemaphoreType.DMA,
    ],
)
def _combine(ys_hbm, dest_hbm, out_hbm, dv, y1, y2, ob, sem1, sem2, semo):
    cid = lax.axis_index("c")
    sid = lax.axis_index("s")
    wid = sid * NC + cid
    n0 = wid * TPW
    nch = TPW // CCH
    DEPTH = 4

    pltpu.sync_copy(dest_hbm.at[wid], dv)

    def fire(i):
        ch, half = i // 2, i % 2
        gp = i % DEPTH
        g1 = pltpu.async_copy(
            ys_hbm.at[dv.at[ch, pl.ds(half * CCH, CCH)]], y1.at[gp], sem1)
        g2 = pltpu.async_copy(
            ys_hbm.at[dv.at[4 + ch, pl.ds(half * CCH, CCH)]], y2.at[gp], sem2)
        return g1, g2

    gath = [fire(i) for i in range(DEPTH - 1)]
    outs = [None] * nch
    for i in range(nch):
        p = i % 2
        gp = i % DEPTH
        if i >= 2:
            outs[i - 2].wait()
        if i + DEPTH - 1 < nch:
            gath.append(fire(i + DEPTH - 1))
        g1, g2 = gath[i]
        g1.wait()
        g2.wait()

        def addrow(r, _):
            himask = jnp.int32(-65536)
            for g in range(OW // L):
                sl = pl.ds(g * L, L)
                w1v = y1[gp, r, sl]
                w2v = y2[gp, r, sl]
                lo = (plsc.bitcast(jnp.left_shift(w1v, 16), jnp.float32)
                      + plsc.bitcast(jnp.left_shift(w2v, 16), jnp.float32))
                hi = (plsc.bitcast(w1v & himask, jnp.float32)
                      + plsc.bitcast(w2v & himask, jnp.float32))
                ob[p, r, sl] = lo
                ob[p, r, pl.ds(OW + g * L, L)] = hi
            return 0

        lax.fori_loop(0, CCH, addrow, 0)
        outs[i] = pltpu.async_copy(
            ob.at[p], out_hbm.at[pl.ds(n0 + i * CCH, CCH), :], semo)
    outs[nch - 2].wait()
    outs[nch - 1].wait()


def kernel(x, wg, fc1_w, fc2_w):
    xf = x.reshape(N, D)
    gate, xp = _gate(xf, wg)
    w1b, w2b = _wcast(fc1_w, fc2_w)
    xs, dest, eid = _route_dispatch(gate, xp)
    ys = _ffn(eid, xs, w1b, w2b)
    out = _combine(ys, dest)
    return out.reshape(B, S, O)
